# baseline (speedup 1.0000x reference)
import functools

import jax
import jax.numpy as jnp
from jax import lax
from jax.experimental import pallas as pl
from jax.experimental.pallas import tpu as pltpu
from jax.experimental.pallas import tpu_sc as plsc

_NC, _NS = 2, 16
_NW = _NC * _NS
_ROWS, _COLS, _OUTC = 16384, 2048, 128
_STRIDE = _COLS // _OUTC
_R = 4
_ROWS_W = _ROWS // _NW
_CHUNKS = _ROWS_W // _R
_NBUF = 8
assert _CHUNKS % _NBUF == 0

_mesh = plsc.VectorSubcoreMesh(core_axis_name="c", subcore_axis_name="s")


@functools.partial(
    pl.kernel,
    out_type=jax.ShapeDtypeStruct((_ROWS, _OUTC), jnp.float32),
    mesh=_mesh,
    scratch_types=[
        [pltpu.VMEM((_R, _COLS), jnp.float32) for _ in range(_NBUF)],
        [pltpu.VMEM((_R, _OUTC), jnp.float32) for _ in range(_NBUF)],
        [pltpu.SemaphoreType.DMA for _ in range(_NBUF)],
        [pltpu.SemaphoreType.DMA for _ in range(_NBUF)],
    ],
    compiler_params=pltpu.CompilerParams(needs_layout_passes=False),
)
def _select_sc(x_hbm, out_hbm, xins, youts, sis, sos):
    wid = lax.axis_index("s") * _NC + lax.axis_index("c")
    row0 = wid * _ROWS_W
    lane = lax.iota(jnp.int32, 16)
    col_sel = lane * _STRIDE

    def in_slice(g):
        return x_hbm.at[pl.ds(row0 + g * _R, _R), :]

    def out_slice(g):
        return out_hbm.at[pl.ds(row0 + g * _R, _R), :]

    for b in range(_NBUF - 1):
        pltpu.async_copy(in_slice(b), xins[b], sis[b])

    def body(h, carry):
        for b in range(_NBUF):
            g = h * _NBUF + b

            @pl.when(g + _NBUF - 1 < _CHUNKS)
            def _():
                nb = (b + _NBUF - 1) % _NBUF
                pltpu.async_copy(in_slice(g + _NBUF - 1), xins[nb], sis[nb])

            pltpu.make_async_copy(in_slice(g), xins[b], sis[b]).wait()

            @pl.when(g >= _NBUF)
            def _():
                pltpu.make_async_copy(youts[b], out_slice(g - _NBUF), sos[b]).wait()

            for r in range(_R):
                rvec = jnp.full((16,), r, jnp.int32)
                for v in range(_OUTC // 16):
                    idx_col = col_sel + (v * 16 * _STRIDE)
                    youts[b][r, pl.ds(v * 16, 16)] = plsc.load_gather(
                        xins[b], [rvec, idx_col]
                    )

            pltpu.async_copy(youts[b], out_slice(g), sos[b])
        return carry

    lax.fori_loop(0, _CHUNKS // _NBUF, body, 0)

    for b in range(_NBUF):
        g = _CHUNKS - _NBUF + b
        pltpu.make_async_copy(youts[b], out_slice(g), sos[b]).wait()


def kernel(x):
    return _select_sc(x)


# speedup vs baseline: 1.0085x; 1.0085x over previous
import functools

import jax
import jax.numpy as jnp
from jax import lax
from jax.experimental import pallas as pl
from jax.experimental.pallas import tpu as pltpu
from jax.experimental.pallas import tpu_sc as plsc

_NC, _NS = 2, 16
_NW = _NC * _NS
_ROWS, _COLS, _OUTC = 16384, 2048, 128
_STRIDE = _COLS // _OUTC
_R = 8
_CSPLIT = 2
_CW = _COLS // _CSPLIT
_OW = _OUTC // _CSPLIT
_ROWS_W = _ROWS // _NW
_CHUNKS = _ROWS_W // _R * _CSPLIT
_NBUF = 8
assert _CHUNKS % _NBUF == 0

_mesh = plsc.VectorSubcoreMesh(core_axis_name="c", subcore_axis_name="s")


@functools.partial(
    pl.kernel,
    out_type=jax.ShapeDtypeStruct((_ROWS, _OUTC), jnp.float32),
    mesh=_mesh,
    scratch_types=[
        [pltpu.VMEM((_R, _CW), jnp.float32) for _ in range(_NBUF)],
        [pltpu.VMEM((_R, _OUTC), jnp.float32) for _ in range(_NBUF // 2)],
        [pltpu.SemaphoreType.DMA for _ in range(_NBUF)],
        [pltpu.SemaphoreType.DMA for _ in range(_NBUF // 2)],
    ],
    compiler_params=pltpu.CompilerParams(needs_layout_passes=False),
)
def _select_sc(x_hbm, out_hbm, xins, youts, sis, sos):
    wid = lax.axis_index("s") * _NC + lax.axis_index("c")
    row0 = wid * _ROWS_W
    lane = lax.iota(jnp.int32, 16)
    col_sel = lane * _STRIDE

    def in_slice(g):
        r = row0 + (g // _CSPLIT) * _R
        return x_hbm.at[pl.ds(r, _R), pl.ds((g % _CSPLIT) * _CW, _CW)]

    def out_slice(p):
        return out_hbm.at[pl.ds(row0 + p * _R, _R), :]

    for b in range(_NBUF - 1):
        pltpu.async_copy(in_slice(b), xins[b], sis[b])

    def body(h, carry):
        for b in range(_NBUF):
            g = h * _NBUF + b
            pltpu.make_async_copy(in_slice(g), xins[b], sis[b]).wait()

            @pl.when(g + _NBUF - 1 < _CHUNKS)
            def _():
                nb = (b + _NBUF - 1) % _NBUF
                pltpu.async_copy(in_slice(g + _NBUF - 1), xins[nb], sis[nb])

            p = h * (_NBUF // 2) + b // 2
            ob = b // 2
            if b % 2 == 0:
                @pl.when(g >= _NBUF)
                def _():
                    pltpu.make_async_copy(
                        youts[ob], out_slice(p - _NBUF // 2), sos[ob]
                    ).wait()

            for r in range(_R):
                rvec = jnp.full((16,), r, jnp.int32)
                for v in range(_OW // 16):
                    idx_col = col_sel + (v * 16 * _STRIDE)
                    youts[ob][r, pl.ds((b % 2) * _OW + v * 16, 16)] = (
                        plsc.load_gather(xins[b], [rvec, idx_col])
                    )

            if b % 2 == 1:
                pltpu.async_copy(youts[ob], out_slice(p), sos[ob])
        return carry

    lax.fori_loop(0, _CHUNKS // _NBUF, body, 0)

    npairs = _CHUNKS // 2
    for ob in range(_NBUF // 2):
        p = npairs - _NBUF // 2 + ob
        pltpu.make_async_copy(youts[ob], out_slice(p), sos[ob]).wait()


def kernel(x):
    return _select_sc(x)


# speedup vs baseline: 1.0103x; 1.0018x over previous
import functools

import jax
import jax.numpy as jnp
from jax import lax
from jax.experimental import pallas as pl
from jax.experimental.pallas import tpu as pltpu
from jax.experimental.pallas import tpu_sc as plsc

_NC, _NS = 2, 16
_NW = _NC * _NS
_ROWS, _COLS, _OUTC = 16384, 2048, 128
_STRIDE = _COLS // _OUTC
_R = 4
_ROWS_W = _ROWS // _NW
_CHUNKS = _ROWS_W // _R
_NBUF = 8
assert _CHUNKS % _NBUF == 0

_mesh = plsc.VectorSubcoreMesh(core_axis_name="c", subcore_axis_name="s")


@functools.partial(
    pl.kernel,
    out_type=jax.ShapeDtypeStruct((_ROWS, _OUTC), jnp.float32),
    mesh=_mesh,
    scratch_types=[
        [pltpu.VMEM((_R, _COLS), jnp.float32) for _ in range(_NBUF)],
        [pltpu.VMEM((_R, _OUTC), jnp.float32) for _ in range(_NBUF)],
        [pltpu.SemaphoreType.DMA for _ in range(_NBUF)],
        [pltpu.SemaphoreType.DMA for _ in range(_NBUF)],
    ],
    compiler_params=pltpu.CompilerParams(needs_layout_passes=False),
)
def _select_sc(x_hbm, out_hbm, xins, youts, sis, sos):
    wid = lax.axis_index("s") * _NC + lax.axis_index("c")
    row0 = wid * _ROWS_W
    lane = lax.iota(jnp.int32, 16)
    col_sel = lane * _STRIDE

    def in_slice(g):
        return x_hbm.at[pl.ds(row0 + g * _R, _R), :]

    def out_slice(g):
        return out_hbm.at[pl.ds(row0 + g * _R, _R), :]

    for b in range(_NBUF - 1):
        pltpu.async_copy(in_slice(b), xins[b], sis[b])

    def body(h, carry):
        for b in range(_NBUF):
            g = h * _NBUF + b
            pltpu.make_async_copy(in_slice(g), xins[b], sis[b]).wait()

            @pl.when(g + _NBUF - 1 < _CHUNKS)
            def _():
                nb = (b + _NBUF - 1) % _NBUF
                pltpu.async_copy(in_slice(g + _NBUF - 1), xins[nb], sis[nb])

            @pl.when(g >= _NBUF)
            def _():
                pltpu.make_async_copy(youts[b], out_slice(g - _NBUF), sos[b]).wait()

            for r in range(_R):
                rvec = jnp.full((16,), r, jnp.int32)
                for v in range(_OUTC // 16):
                    idx_col = col_sel + (v * 16 * _STRIDE)
                    youts[b][r, pl.ds(v * 16, 16)] = plsc.load_gather(
                        xins[b], [rvec, idx_col]
                    )

            pltpu.async_copy(youts[b], out_slice(g), sos[b])
        return carry

    lax.fori_loop(0, _CHUNKS // _NBUF, body, 0)

    for b in range(_NBUF):
        g = _CHUNKS - _NBUF + b
        pltpu.make_async_copy(youts[b], out_slice(g), sos[b]).wait()


def kernel(x):
    return _select_sc(x)
